# baseline (speedup 1.0000x reference)
import jax
import jax.numpy as jnp
from jax import lax
from jax.experimental import pallas as pl
from jax.experimental.pallas import tpu as pltpu
from jax.experimental.pallas import tpu_sc as plsc

EMBED_DIM = 32
NUM_CORES = 2
NUM_SUBCORES = 16
NW = NUM_CORES * NUM_SUBCORES
CHUNK = 128
K = 5
NBUF = 3
PHASE_ROWS = K * CHUNK


def _sc_embed_add(table, xf, cf, d0, d1_size):
    n_chunks = xf.shape[1]
    n_phases = n_chunks // K
    mesh = plsc.VectorSubcoreMesh(core_axis_name="c", subcore_axis_name="s")

    def body(table_hbm, x_hbm, c_hbm, out_hbm, xv, cv, bufs, semg, semo):
        wid = lax.axis_index("s") * NUM_CORES + lax.axis_index("c")
        pltpu.sync_copy(x_hbm.at[wid], xv)
        pltpu.sync_copy(c_hbm.at[wid], cv)

        def drain_gathers(b):
            pltpu.make_async_copy(
                table_hbm.at[pl.ds(0, PHASE_ROWS)], bufs.at[b], semg.at[b]
            ).wait()

        def drain_out(b):
            def w(t, c):
                pltpu.make_async_copy(
                    bufs.at[0, pl.ds(0, CHUNK), pl.ds(0, 1)],
                    out_hbm.at[0],
                    semo.at[b],
                ).wait()
                return c
            lax.fori_loop(0, K * EMBED_DIM, w, 0)

        def step(p, carry):
            b_a = lax.rem(p, NBUF)
            b_b = lax.rem(p + (NBUF - 1), NBUF)
            b_c = lax.rem(p + (NBUF - 2), NBUF)

            @pl.when(jnp.logical_and(p >= 2, p <= n_phases + 1))
            def _():
                drain_gathers(b_c)
                for i in range(K):
                    pair = wid * n_chunks + (p - 2) * K + i
                    d1 = lax.div(pair, NW)
                    j = lax.rem(pair, NW)
                    row0 = d1 * 1024 + j * 8

                    def w(d2, c):
                        ti = lax.shift_right_logical(d2, 3)
                        b8 = lax.rem(d2, 8)
                        pltpu.async_copy(
                            bufs.at[b_c, pl.ds(i * CHUNK, CHUNK), pl.ds(d2, 1)],
                            out_hbm.at[row0 + ti * 256 + b8],
                            semo.at[b_c],
                        )
                        return c

                    lax.fori_loop(0, EMBED_DIM, w, 0)

            @pl.when(jnp.logical_and(p >= 1, p <= n_phases))
            def _():
                drain_gathers(b_b)
                for i in range(K):
                    pltpu.async_copy(
                        table_hbm.at[cv.at[(p - 1) * K + i]],
                        bufs.at[b_b].at[pl.ds(i * CHUNK, CHUNK)],
                        semg.at[b_b],
                        add=True,
                    )

            @pl.when(p <= n_phases - 1)
            def _():
                @pl.when(p >= NBUF)
                def _():
                    drain_out(b_a)

                for i in range(K):
                    pltpu.async_copy(
                        table_hbm.at[xv.at[p * K + i]],
                        bufs.at[b_a].at[pl.ds(i * CHUNK, CHUNK)],
                        semg.at[b_a],
                    )

            return carry

        lax.fori_loop(0, n_phases + 2, step, 0)
        for b in range(NBUF):
            drain_out(b)

    run = pl.kernel(
        body,
        out_type=jax.ShapeDtypeStruct((d0 * d1_size * EMBED_DIM // CHUNK, CHUNK, 1), jnp.float32),
        mesh=mesh,
        scratch_types=[
            pltpu.VMEM((n_chunks, CHUNK), jnp.int32),
            pltpu.VMEM((n_chunks, CHUNK), jnp.int32),
            pltpu.VMEM((NBUF, PHASE_ROWS, EMBED_DIM), jnp.float32),
            pltpu.SemaphoreType.DMA((NBUF,)),
            pltpu.SemaphoreType.DMA((NBUF,)),
        ],
        compiler_params=pltpu.CompilerParams(use_tc_tiling_on_sc=False),
    )
    return run(table, xf, cf)


def kernel(x, constant, table):
    d0, d1 = x.shape
    xf = x.T.reshape(NW, d1, CHUNK).astype(jnp.int32)
    cf = constant.T.reshape(NW, d1, CHUNK).astype(jnp.int32)
    out3 = _sc_embed_add(table, xf, cf, d0, d1)
    out5 = out3.reshape(d1, 4, d0 // CHUNK, 8, CHUNK)
    return out5.transpose(2, 4, 0, 1, 3).reshape(d0, d1, EMBED_DIM)


# speedup vs baseline: 24.7767x; 24.7767x over previous
import jax
import jax.numpy as jnp
from jax import lax
from jax.experimental import pallas as pl
from jax.experimental.pallas import tpu as pltpu
from jax.experimental.pallas import tpu_sc as plsc

EMBED_DIM = 32
NUM_CORES = 2
NUM_SUBCORES = 16
NW = NUM_CORES * NUM_SUBCORES
CHUNK = 128
K = 5
NBUF = 3
PHASE_ROWS = K * CHUNK


def _sc_embed_add(table, xf, cf, d0, d1_size):
    n_chunks = xf.shape[1]
    n_phases = n_chunks // K
    mesh = plsc.VectorSubcoreMesh(core_axis_name="c", subcore_axis_name="s")

    def body(table_hbm, x_hbm, c_hbm, out_hbm, xv, cv, bufs, semg, semo):
        wid = lax.axis_index("s") * NUM_CORES + lax.axis_index("c")
        pltpu.sync_copy(x_hbm.at[wid], xv)
        pltpu.sync_copy(c_hbm.at[wid], cv)

        def drain_gathers(b):
            pltpu.make_async_copy(
                table_hbm.at[pl.ds(0, PHASE_ROWS)], bufs.at[b], semg.at[b]
            ).wait()

        def drain_out(b):
            pltpu.make_async_copy(
                bufs.at[b], out_hbm.at[pl.ds(0, PHASE_ROWS), 0], semo.at[b]
            ).wait()

        def step(p, carry):
            b_a = lax.rem(p, NBUF)
            b_b = lax.rem(p + (NBUF - 1), NBUF)
            b_c = lax.rem(p + (NBUF - 2), NBUF)

            @pl.when(jnp.logical_and(p >= 2, p <= n_phases + 1))
            def _():
                drain_gathers(b_c)
                for i in range(K):
                    pair = wid * n_chunks + (p - 2) * K + i
                    d1 = lax.div(pair, NW)
                    j = lax.rem(pair, NW)
                    pltpu.async_copy(
                        bufs.at[b_c].at[pl.ds(i * CHUNK, CHUNK)],
                        out_hbm.at[pl.ds(j * CHUNK, CHUNK), d1],
                        semo.at[b_c],
                    )

            @pl.when(jnp.logical_and(p >= 1, p <= n_phases))
            def _():
                drain_gathers(b_b)
                for i in range(K):
                    pltpu.async_copy(
                        table_hbm.at[cv.at[(p - 1) * K + i]],
                        bufs.at[b_b].at[pl.ds(i * CHUNK, CHUNK)],
                        semg.at[b_b],
                        add=True,
                    )

            @pl.when(p <= n_phases - 1)
            def _():
                @pl.when(p >= NBUF)
                def _():
                    drain_out(b_a)

                for i in range(K):
                    pltpu.async_copy(
                        table_hbm.at[xv.at[p * K + i]],
                        bufs.at[b_a].at[pl.ds(i * CHUNK, CHUNK)],
                        semg.at[b_a],
                    )

            return carry

        lax.fori_loop(0, n_phases + 2, step, 0)
        for b in range(NBUF):
            drain_out(b)

    run = pl.kernel(
        body,
        out_type=jax.ShapeDtypeStruct((d0, d1_size, EMBED_DIM), jnp.float32),
        mesh=mesh,
        scratch_types=[
            pltpu.VMEM((n_chunks, CHUNK), jnp.int32),
            pltpu.VMEM((n_chunks, CHUNK), jnp.int32),
            pltpu.VMEM((NBUF, PHASE_ROWS, EMBED_DIM), jnp.float32),
            pltpu.SemaphoreType.DMA((NBUF,)),
            pltpu.SemaphoreType.DMA((NBUF,)),
        ],
        compiler_params=pltpu.CompilerParams(use_tc_tiling_on_sc=False),
    )
    return run(table, xf, cf)


def kernel(x, constant, table):
    d0, d1 = x.shape
    xf = x.T.reshape(NW, d1, CHUNK).astype(jnp.int32)
    cf = constant.T.reshape(NW, d1, CHUNK).astype(jnp.int32)
    return _sc_embed_add(table, xf, cf, d0, d1)


# speedup vs baseline: 24.8955x; 1.0048x over previous
import jax
import jax.numpy as jnp
from jax import lax
from jax.experimental import pallas as pl
from jax.experimental.pallas import tpu as pltpu
from jax.experimental.pallas import tpu_sc as plsc

EMBED_DIM = 32
NUM_CORES = 2
NUM_SUBCORES = 16
NW = NUM_CORES * NUM_SUBCORES
CHUNK = 128
K = 5
NBUF = 3
PHASE_ROWS = K * CHUNK
CHUNK_ELEMS = CHUNK * EMBED_DIM
PHASE_ELEMS = K * CHUNK_ELEMS
TILE_ELEMS = 8 * CHUNK


def _sc_embed_add(table, xf, cf, n_elems):
    n_chunks = xf.shape[1]
    n_phases = n_chunks // K
    d1_stride = 4 * NW * TILE_ELEMS
    mesh = plsc.VectorSubcoreMesh(core_axis_name="c", subcore_axis_name="s")

    def body(table_hbm, x_hbm, c_hbm, out_hbm, xv, cv, bufs, buft, semg, semo):
        wid = lax.axis_index("s") * NUM_CORES + lax.axis_index("c")
        pltpu.sync_copy(x_hbm.at[wid], xv)
        pltpu.sync_copy(c_hbm.at[wid], cv)
        lanes = lax.iota(jnp.int32, 16)

        def drain_gathers(b):
            pltpu.make_async_copy(
                table_hbm.at[pl.ds(0, PHASE_ROWS)], bufs.at[b], semg.at[b]
            ).wait()

        def drain_writes(m):
            pltpu.make_async_copy(
                buft.at[m], out_hbm.at[pl.ds(0, PHASE_ELEMS)], semo.at[m]
            ).wait()

        def step(p, carry):
            b_a = lax.rem(p, NBUF)
            b_b = lax.rem(p + (NBUF - 1), NBUF)
            b_c = lax.rem(p + (NBUF - 2), NBUF)
            m = lax.rem(p, 2)

            @pl.when(jnp.logical_and(p >= 2, p <= n_phases + 1))
            def _():
                drain_gathers(b_c)

                @pl.when(p >= 4)
                def _():
                    drain_writes(m)

                for i in range(K):
                    pair = wid * n_chunks + (p - 2) * K + i
                    d1 = lax.div(pair, NW)
                    j = lax.rem(pair, NW)

                    @plsc.parallel_loop(0, 256, unroll=8)
                    def _(t):
                        d2 = lax.shift_right_logical(t, 3)
                        s = lax.rem(t, 8)
                        rows = lanes + (i * CHUNK + 16 * s)
                        cols = jnp.broadcast_to(d2, (16,)).astype(jnp.int32)
                        v = plsc.load_gather(bufs.at[b_c], [rows, cols])
                        buft[m, pl.ds(i * CHUNK_ELEMS + d2 * CHUNK + 16 * s, 16)] = v

                    base = d1 * d1_stride + j * TILE_ELEMS
                    for a in range(4):
                        pltpu.async_copy(
                            buft.at[m, pl.ds(i * CHUNK_ELEMS + a * TILE_ELEMS, TILE_ELEMS)],
                            out_hbm.at[pl.ds(base + a * NW * TILE_ELEMS, TILE_ELEMS)],
                            semo.at[m],
                        )

            @pl.when(jnp.logical_and(p >= 1, p <= n_phases))
            def _():
                drain_gathers(b_b)
                for i in range(K):
                    pltpu.async_copy(
                        table_hbm.at[cv.at[(p - 1) * K + i]],
                        bufs.at[b_b].at[pl.ds(i * CHUNK, CHUNK)],
                        semg.at[b_b],
                        add=True,
                    )

            @pl.when(p <= n_phases - 1)
            def _():
                for i in range(K):
                    pltpu.async_copy(
                        table_hbm.at[xv.at[p * K + i]],
                        bufs.at[b_a].at[pl.ds(i * CHUNK, CHUNK)],
                        semg.at[b_a],
                    )

            return carry

        lax.fori_loop(0, n_phases + 2, step, 0)
        for m in range(2):
            drain_writes(m)

    run = pl.kernel(
        body,
        out_type=jax.ShapeDtypeStruct((n_elems,), jnp.float32),
        mesh=mesh,
        scratch_types=[
            pltpu.VMEM((n_chunks, CHUNK), jnp.int32),
            pltpu.VMEM((n_chunks, CHUNK), jnp.int32),
            pltpu.VMEM((NBUF, PHASE_ROWS, EMBED_DIM), jnp.float32),
            pltpu.VMEM((2, PHASE_ELEMS), jnp.float32),
            pltpu.SemaphoreType.DMA((NBUF,)),
            pltpu.SemaphoreType.DMA((2,)),
        ],
        compiler_params=pltpu.CompilerParams(
            use_tc_tiling_on_sc=False, needs_layout_passes=False
        ),
    )
    return run(table, xf, cf)


def kernel(x, constant, table):
    d0, d1 = x.shape
    xf = x.T.reshape(NW, d1, CHUNK).astype(jnp.int32)
    cf = constant.T.reshape(NW, d1, CHUNK).astype(jnp.int32)
    flat = _sc_embed_add(table, xf, cf, x.size * EMBED_DIM)
    out5 = flat.reshape(d1, 4, d0 // CHUNK, 8, CHUNK)
    return out5.transpose(2, 4, 0, 1, 3).reshape(d0, d1, EMBED_DIM)


# speedup vs baseline: 24.9166x; 1.0008x over previous
import jax
import jax.numpy as jnp
from jax import lax
from jax.experimental import pallas as pl
from jax.experimental.pallas import tpu as pltpu
from jax.experimental.pallas import tpu_sc as plsc

EMBED_DIM = 32
NUM_CORES = 2
NUM_SUBCORES = 16
NW = NUM_CORES * NUM_SUBCORES
CHUNK = 128
K = 5
NBUF = 3
PHASE_ROWS = K * CHUNK
CHUNK_ELEMS = CHUNK * EMBED_DIM
PHASE_ELEMS = K * CHUNK_ELEMS
TILE_ELEMS = 8 * CHUNK


def _sc_embed_add(table, xf, cf, n_elems):
    n_chunks = xf.shape[1]
    n_phases = n_chunks // K
    d1_stride = 4 * NW * TILE_ELEMS
    mesh = plsc.VectorSubcoreMesh(core_axis_name="c", subcore_axis_name="s")

    def body(table_hbm, x_hbm, c_hbm, out_hbm, xv, cv, bufs, buft, semg, semo):
        wid = lax.axis_index("s") * NUM_CORES + lax.axis_index("c")
        pltpu.sync_copy(x_hbm.at[wid], xv)
        pltpu.sync_copy(c_hbm.at[wid], cv)
        lanes = lax.iota(jnp.int32, 16)

        def drain_gathers(b):
            pltpu.make_async_copy(
                table_hbm.at[pl.ds(0, PHASE_ROWS)], bufs.at[b], semg.at[b]
            ).wait()

        def drain_writes(m):
            pltpu.make_async_copy(
                buft.at[m], out_hbm.at[pl.ds(0, PHASE_ELEMS)], semo.at[m]
            ).wait()

        def step(p, carry):
            b_a = lax.rem(p, NBUF)
            b_b = lax.rem(p + (NBUF - 1), NBUF)
            b_c = lax.rem(p + (NBUF - 2), NBUF)
            m = lax.rem(p, 2)

            @pl.when(jnp.logical_and(p >= 2, p <= n_phases + 1))
            def _():
                drain_gathers(b_c)

                @pl.when(p >= 4)
                def _():
                    drain_writes(m)

                chunk_rows = bufs.at[b_c]
                for i in range(K):
                    pair = wid * n_chunks + (p - 2) * K + i
                    d1 = lax.div(pair, NW)
                    j = lax.rem(pair, NW)

                    @plsc.parallel_loop(0, 256, unroll=16)
                    def _(t):
                        d2 = lax.shift_right_logical(t, 3)
                        s16 = lax.shift_left(jnp.bitwise_and(t, 7), 4)
                        rows = lanes + (i * CHUNK + s16)
                        cols = jnp.broadcast_to(d2, (16,)).astype(jnp.int32)
                        v = plsc.load_gather(chunk_rows, [rows, cols])
                        buft[m, pl.ds(i * CHUNK_ELEMS + lax.shift_left(d2, 7) + s16, 16)] = v

                    base = d1 * d1_stride + j * TILE_ELEMS
                    for a in range(4):
                        pltpu.async_copy(
                            buft.at[m, pl.ds(i * CHUNK_ELEMS + a * TILE_ELEMS, TILE_ELEMS)],
                            out_hbm.at[pl.ds(base + a * NW * TILE_ELEMS, TILE_ELEMS)],
                            semo.at[m],
                        )

            @pl.when(jnp.logical_and(p >= 1, p <= n_phases))
            def _():
                drain_gathers(b_b)
                for i in range(K):
                    pltpu.async_copy(
                        table_hbm.at[cv.at[(p - 1) * K + i]],
                        bufs.at[b_b].at[pl.ds(i * CHUNK, CHUNK)],
                        semg.at[b_b],
                        add=True,
                    )

            @pl.when(p <= n_phases - 1)
            def _():
                for i in range(K):
                    pltpu.async_copy(
                        table_hbm.at[xv.at[p * K + i]],
                        bufs.at[b_a].at[pl.ds(i * CHUNK, CHUNK)],
                        semg.at[b_a],
                    )

            return carry

        lax.fori_loop(0, n_phases + 2, step, 0)
        for m in range(2):
            drain_writes(m)

    run = pl.kernel(
        body,
        out_type=jax.ShapeDtypeStruct((n_elems,), jnp.float32),
        mesh=mesh,
        scratch_types=[
            pltpu.VMEM((n_chunks, CHUNK), jnp.int32),
            pltpu.VMEM((n_chunks, CHUNK), jnp.int32),
            pltpu.VMEM((NBUF, PHASE_ROWS, EMBED_DIM), jnp.float32),
            pltpu.VMEM((2, PHASE_ELEMS), jnp.float32),
            pltpu.SemaphoreType.DMA((NBUF,)),
            pltpu.SemaphoreType.DMA((2,)),
        ],
        compiler_params=pltpu.CompilerParams(
            use_tc_tiling_on_sc=False, needs_layout_passes=False
        ),
    )
    return run(table, xf, cf)


def kernel(x, constant, table):
    d0, d1 = x.shape
    xf = x.T.reshape(NW, d1, CHUNK).astype(jnp.int32)
    cf = constant.T.reshape(NW, d1, CHUNK).astype(jnp.int32)
    flat = _sc_embed_add(table, xf, cf, x.size * EMBED_DIM)
    out5 = flat.reshape(d1, 4, d0 // CHUNK, 8, CHUNK)
    return out5.transpose(2, 4, 0, 1, 3).reshape(d0, d1, EMBED_DIM)


# speedup vs baseline: 26.2177x; 1.0522x over previous
import jax
import jax.numpy as jnp
from jax import lax
from jax.experimental import pallas as pl
from jax.experimental.pallas import tpu as pltpu
from jax.experimental.pallas import tpu_sc as plsc

EMBED_DIM = 32
NUM_CORES = 2
NUM_SUBCORES = 16
NW = NUM_CORES * NUM_SUBCORES
CHUNK = 128
K = 5
NBUF = 3
PHASE_ROWS = K * CHUNK
CHUNK_ELEMS = CHUNK * EMBED_DIM
PHASE_ELEMS = K * CHUNK_ELEMS
TILE_ELEMS = 8 * CHUNK


def _sc_embed_add(table, xf, cf, n_elems):
    n_chunks = xf.shape[1]
    n_phases = n_chunks // K
    d1_stride = 4 * NW * TILE_ELEMS
    mesh = plsc.VectorSubcoreMesh(core_axis_name="c", subcore_axis_name="s")

    def body(table_hbm, x_hbm, c_hbm, out_hbm, xv, cv, bufs, buft, semg, semo):
        wid = lax.axis_index("s") * NUM_CORES + lax.axis_index("c")
        pltpu.sync_copy(x_hbm.at[wid], xv)
        pltpu.sync_copy(c_hbm.at[wid], cv)
        lanes = lax.iota(jnp.int32, 16)
        rows8 = [lanes + 16 * sv for sv in range(8)]

        def drain_gathers(b):
            pltpu.make_async_copy(
                table_hbm.at[pl.ds(0, PHASE_ROWS)], bufs.at[b], semg.at[b]
            ).wait()

        def drain_writes(m):
            pltpu.make_async_copy(
                buft.at[m], out_hbm.at[pl.ds(0, PHASE_ELEMS)], semo.at[m]
            ).wait()

        def step(p, carry):
            b_a = lax.rem(p, NBUF)
            b_b = lax.rem(p + (NBUF - 1), NBUF)
            b_c = lax.rem(p + (NBUF - 2), NBUF)
            m = lax.rem(p, 2)

            @pl.when(jnp.logical_and(p >= 2, p <= n_phases + 1))
            def _():
                drain_gathers(b_c)

                @pl.when(p >= 4)
                def _():
                    drain_writes(m)

                for i in range(K):
                    pair = wid * n_chunks + (p - 2) * K + i
                    d1 = lax.div(pair, NW)
                    j = lax.rem(pair, NW)
                    chunk_slice = bufs.at[b_c].at[pl.ds(i * CHUNK, CHUNK)]

                    @plsc.parallel_loop(0, EMBED_DIM, unroll=4)
                    def _(d2):
                        cols = jnp.broadcast_to(d2, (16,)).astype(jnp.int32)
                        off = i * CHUNK_ELEMS + lax.shift_left(d2, 7)
                        for sv in range(8):
                            v = plsc.load_gather(chunk_slice, [rows8[sv], cols])
                            buft[m, pl.ds(off + 16 * sv, 16)] = v

                    base = d1 * d1_stride + j * TILE_ELEMS
                    for a in range(4):
                        pltpu.async_copy(
                            buft.at[m, pl.ds(i * CHUNK_ELEMS + a * TILE_ELEMS, TILE_ELEMS)],
                            out_hbm.at[pl.ds(base + a * NW * TILE_ELEMS, TILE_ELEMS)],
                            semo.at[m],
                        )

            @pl.when(jnp.logical_and(p >= 1, p <= n_phases))
            def _():
                drain_gathers(b_b)
                for i in range(K):
                    pltpu.async_copy(
                        table_hbm.at[cv.at[(p - 1) * K + i]],
                        bufs.at[b_b].at[pl.ds(i * CHUNK, CHUNK)],
                        semg.at[b_b],
                        add=True,
                    )

            @pl.when(p <= n_phases - 1)
            def _():
                for i in range(K):
                    pltpu.async_copy(
                        table_hbm.at[xv.at[p * K + i]],
                        bufs.at[b_a].at[pl.ds(i * CHUNK, CHUNK)],
                        semg.at[b_a],
                    )

            return carry

        lax.fori_loop(0, n_phases + 2, step, 0)
        for m in range(2):
            drain_writes(m)

    run = pl.kernel(
        body,
        out_type=jax.ShapeDtypeStruct((n_elems,), jnp.float32),
        mesh=mesh,
        scratch_types=[
            pltpu.VMEM((n_chunks, CHUNK), jnp.int32),
            pltpu.VMEM((n_chunks, CHUNK), jnp.int32),
            pltpu.VMEM((NBUF, PHASE_ROWS, EMBED_DIM), jnp.float32),
            pltpu.VMEM((2, PHASE_ELEMS), jnp.float32),
            pltpu.SemaphoreType.DMA((NBUF,)),
            pltpu.SemaphoreType.DMA((2,)),
        ],
        compiler_params=pltpu.CompilerParams(
            use_tc_tiling_on_sc=False, needs_layout_passes=False
        ),
    )
    return run(table, xf, cf)


def kernel(x, constant, table):
    d0, d1 = x.shape
    xf = x.T.reshape(NW, d1, CHUNK).astype(jnp.int32)
    cf = constant.T.reshape(NW, d1, CHUNK).astype(jnp.int32)
    flat = _sc_embed_add(table, xf, cf, x.size * EMBED_DIM)
    out5 = flat.reshape(d1, 4, d0 // CHUNK, 8, CHUNK)
    return out5.transpose(2, 4, 0, 1, 3).reshape(d0, d1, EMBED_DIM)


# speedup vs baseline: 26.8156x; 1.0228x over previous
import jax
import jax.numpy as jnp
from jax import lax
from jax.experimental import pallas as pl
from jax.experimental.pallas import tpu as pltpu
from jax.experimental.pallas import tpu_sc as plsc

EMBED_DIM = 32
NUM_CORES = 2
NUM_SUBCORES = 16
NW = NUM_CORES * NUM_SUBCORES
CHUNK = 128
K = 5
NBUF = 3
PHASE_ROWS = K * CHUNK
CHUNK_ELEMS = CHUNK * EMBED_DIM
PHASE_ELEMS = K * CHUNK_ELEMS
TILE_ELEMS = 8 * CHUNK


def _sc_embed_add(table, xf, cf, n_elems):
    n_chunks = xf.shape[1]
    n_phases = n_chunks // K
    d1_stride = 4 * NW * TILE_ELEMS
    mesh = plsc.VectorSubcoreMesh(core_axis_name="c", subcore_axis_name="s")

    def body(table_hbm, x_hbm, c_hbm, out_hbm, xv, cv, bufs, buft, semg, semo):
        wid = lax.axis_index("s") * NUM_CORES + lax.axis_index("c")
        pltpu.sync_copy(x_hbm.at[wid], xv)
        pltpu.sync_copy(c_hbm.at[wid], cv)
        lanes = lax.iota(jnp.int32, 16)
        rows8 = [lanes + 16 * sv for sv in range(8)]

        def drain_gathers(b):
            pltpu.make_async_copy(
                table_hbm.at[pl.ds(0, PHASE_ROWS)], bufs.at[b], semg.at[b]
            ).wait()

        def drain_writes(m):
            pltpu.make_async_copy(
                buft.at[m], out_hbm.at[pl.ds(0, PHASE_ELEMS)], semo.at[m]
            ).wait()

        def step(p, carry):
            b_a = lax.rem(p, NBUF)
            b_b = lax.rem(p + (NBUF - 1), NBUF)
            b_c = lax.rem(p + (NBUF - 2), NBUF)
            m = lax.rem(p, 2)

            @pl.when(jnp.logical_and(p >= 1, p <= n_phases))
            def _():
                drain_gathers(b_b)
                for i in range(K):
                    pltpu.async_copy(
                        table_hbm.at[cv.at[(p - 1) * K + i]],
                        bufs.at[b_b].at[pl.ds(i * CHUNK, CHUNK)],
                        semg.at[b_b],
                        add=True,
                    )

            @pl.when(p <= n_phases - 1)
            def _():
                for i in range(K):
                    pltpu.async_copy(
                        table_hbm.at[xv.at[p * K + i]],
                        bufs.at[b_a].at[pl.ds(i * CHUNK, CHUNK)],
                        semg.at[b_a],
                    )

            @pl.when(jnp.logical_and(p >= 2, p <= n_phases + 1))
            def _():
                drain_gathers(b_c)

                @pl.when(p >= 4)
                def _():
                    drain_writes(m)

                for i in range(K):
                    pair = wid * n_chunks + (p - 2) * K + i
                    d1 = lax.div(pair, NW)
                    j = lax.rem(pair, NW)
                    chunk_slice = bufs.at[b_c].at[pl.ds(i * CHUNK, CHUNK)]

                    @plsc.parallel_loop(0, EMBED_DIM, unroll=8)
                    def _(d2):
                        cols = jnp.broadcast_to(d2, (16,)).astype(jnp.int32)
                        off = i * CHUNK_ELEMS + lax.shift_left(d2, 7)
                        for sv in range(8):
                            v = plsc.load_gather(chunk_slice, [rows8[sv], cols])
                            buft[m, pl.ds(off + 16 * sv, 16)] = v

                    base = d1 * d1_stride + j * TILE_ELEMS
                    for a in range(4):
                        pltpu.async_copy(
                            buft.at[m, pl.ds(i * CHUNK_ELEMS + a * TILE_ELEMS, TILE_ELEMS)],
                            out_hbm.at[pl.ds(base + a * NW * TILE_ELEMS, TILE_ELEMS)],
                            semo.at[m],
                        )

            return carry

        lax.fori_loop(0, n_phases + 2, step, 0)
        for m in range(2):
            drain_writes(m)

    run = pl.kernel(
        body,
        out_type=jax.ShapeDtypeStruct((n_elems,), jnp.float32),
        mesh=mesh,
        scratch_types=[
            pltpu.VMEM((n_chunks, CHUNK), jnp.int32),
            pltpu.VMEM((n_chunks, CHUNK), jnp.int32),
            pltpu.VMEM((NBUF, PHASE_ROWS, EMBED_DIM), jnp.float32),
            pltpu.VMEM((2, PHASE_ELEMS), jnp.float32),
            pltpu.SemaphoreType.DMA((NBUF,)),
            pltpu.SemaphoreType.DMA((2,)),
        ],
        compiler_params=pltpu.CompilerParams(
            use_tc_tiling_on_sc=False, needs_layout_passes=False
        ),
    )
    return run(table, xf, cf)


def kernel(x, constant, table):
    d0, d1 = x.shape
    xf = x.T.reshape(NW, d1, CHUNK).astype(jnp.int32)
    cf = constant.T.reshape(NW, d1, CHUNK).astype(jnp.int32)
    flat = _sc_embed_add(table, xf, cf, x.size * EMBED_DIM)
    out5 = flat.reshape(d1, 4, d0 // CHUNK, 8, CHUNK)
    return out5.transpose(2, 4, 0, 1, 3).reshape(d0, d1, EMBED_DIM)
